# baseline (speedup 1.0000x reference)
import functools

import jax
import jax.numpy as jnp
from jax import lax
from jax.experimental import pallas as pl
from jax.experimental.pallas import tpu as pltpu
from jax.experimental.pallas import tpu_sc as plsc

_NC = 2
_NS = 16
_NW = _NC * _NS
_LB = 128


def _deg_kernel_fn(n_nodes, n_batches, rows_sh):
    rows_per_sub = rows_sh // _NS
    zcopies = rows_per_sub // _LB

    def body(dst_flat, part_flat, dst_v, ones_v, zbuf, deg_sh):
        cid = lax.axis_index("c")
        sid = lax.axis_index("s")
        wid = sid * _NC + cid
        dst3 = dst_flat
        part = part_flat

        zero16 = jnp.zeros((16,), jnp.float32)
        one16 = jnp.ones((16,), jnp.float32)

        def fill(i, _):
            zbuf[i, 0:16] = zero16
            ones_v[i, 0:16] = one16
            return 0

        lax.fori_loop(0, _LB, fill, 0)

        for k in range(zcopies):
            pltpu.sync_copy(zbuf, deg_sh.at[pl.ds(sid * rows_per_sub + k * _LB, _LB)])
        plsc.subcore_barrier()

        pltpu.sync_copy(dst3.at[pl.ds(wid * n_batches, n_batches)], dst_v)

        def step(j, _):
            pltpu.sync_copy(ones_v, deg_sh.at[dst_v.at[j]], add=True)
            return 0

        lax.fori_loop(0, n_batches, step, 0)
        plsc.subcore_barrier()

        pltpu.sync_copy(
            deg_sh.at[pl.ds(sid * rows_per_sub, rows_per_sub)],
            part.at[pl.ds(cid * rows_sh + sid * rows_per_sub, rows_per_sub)],
        )

    return pl.kernel(
        body,
        out_type=jax.ShapeDtypeStruct((_NC * rows_sh, 16), jnp.float32),
        mesh=plsc.VectorSubcoreMesh(core_axis_name="c", subcore_axis_name="s"),
        scratch_types=[
            pltpu.VMEM((n_batches, _LB), jnp.int32),
            pltpu.VMEM((_LB, 16), jnp.float32),
            pltpu.VMEM((_LB, 16), jnp.float32),
            pltpu.VMEM_SHARED((rows_sh, 16), jnp.float32),
        ],
        compiler_params=pltpu.CompilerParams(use_tc_tiling_on_sc=False),
    )


def _agg_kernel_fn(n_nodes, d, n_batches, rows_sh):
    rows_per_sub = rows_sh // _NS
    zcopies = rows_per_sub // _LB

    nbuf = 8
    assert n_batches % nbuf == 0 and d % 32 == 0
    dh = d // 2
    n_per_sub = n_nodes // _NS

    def body(h128, src3, dst3, part_lo, part_hi, src_v, dst_v, rows_v,
             zbuf, sg0, sg1, sg2, sg3, sg4, sg5, sg6, sg7,
             ss0, ss1, ss2, ss3, ss4, ss5, ss6, ss7,
             agg_sh, hlo_sh, hhi_sh):
        cid = lax.axis_index("c")
        sid = lax.axis_index("s")
        wid = sid * _NC + cid
        sgs = (sg0, sg1, sg2, sg3, sg4, sg5, sg6, sg7)
        sss = (ss0, ss1, ss2, ss3, ss4, ss5, ss6, ss7)

        zero16 = jnp.zeros((16,), jnp.float32)

        def fill(i, _):
            for j in range(dh // 16):
                zbuf[i, pl.ds(j * 16, 16)] = zero16
            return 0

        lax.fori_loop(0, _LB, fill, 0)

        pltpu.sync_copy(
            h128.at[pl.ds(sid * n_per_sub, n_per_sub), pl.ds(0, dh)],
            hlo_sh.at[pl.ds(sid * n_per_sub, n_per_sub)],
        )
        pltpu.sync_copy(
            h128.at[pl.ds(sid * n_per_sub, n_per_sub), pl.ds(dh, dh)],
            hhi_sh.at[pl.ds(sid * n_per_sub, n_per_sub)],
        )

        pltpu.sync_copy(src3.at[pl.ds(wid * n_batches, n_batches)], src_v)
        pltpu.sync_copy(dst3.at[pl.ds(wid * n_batches, n_batches)], dst_v)

        def phase(h_sh, part):
            for k in range(zcopies):
                pltpu.sync_copy(
                    zbuf, agg_sh.at[pl.ds(sid * rows_per_sub + k * _LB, _LB)]
                )
            plsc.subcore_barrier()

            for b in range(nbuf):
                pltpu.async_copy(h_sh.at[src_v.at[b]], rows_v.at[b], sgs[b])

            def group(jj, _):
                for b in range(nbuf):
                    j = jj * nbuf + b
                    pltpu.make_async_copy(
                        h_sh.at[src_v.at[j]], rows_v.at[b], sgs[b]
                    ).wait()
                    pltpu.async_copy(rows_v.at[b], agg_sh.at[dst_v.at[j]], sss[b], add=True)
                for b in range(nbuf):
                    j = jj * nbuf + b
                    pltpu.make_async_copy(
                        rows_v.at[b], agg_sh.at[dst_v.at[j]], sss[b]
                    ).wait()

                    @pl.when(j + nbuf < n_batches)
                    def _():
                        pltpu.async_copy(
                            h_sh.at[src_v.at[j + nbuf]], rows_v.at[b], sgs[b]
                        )

                return 0

            lax.fori_loop(0, n_batches // nbuf, group, 0)
            plsc.subcore_barrier()

            pltpu.sync_copy(
                agg_sh.at[pl.ds(sid * rows_per_sub, rows_per_sub)],
                part.at[pl.ds(cid * rows_sh + sid * rows_per_sub, rows_per_sub)],
            )

        phase(hlo_sh, part_lo)
        phase(hhi_sh, part_hi)

    return pl.kernel(
        body,
        out_type=(
            jax.ShapeDtypeStruct((_NC * rows_sh, dh), jnp.float32),
            jax.ShapeDtypeStruct((_NC * rows_sh, dh), jnp.float32),
        ),
        mesh=plsc.VectorSubcoreMesh(core_axis_name="c", subcore_axis_name="s"),
        scratch_types=[
            pltpu.VMEM((n_batches, _LB), jnp.int32),
            pltpu.VMEM((n_batches, _LB), jnp.int32),
            pltpu.VMEM((nbuf, _LB, dh), jnp.float32),
            pltpu.VMEM((_LB, dh), jnp.float32),
            pltpu.SemaphoreType.DMA,
            pltpu.SemaphoreType.DMA,
            pltpu.SemaphoreType.DMA,
            pltpu.SemaphoreType.DMA,
            pltpu.SemaphoreType.DMA,
            pltpu.SemaphoreType.DMA,
            pltpu.SemaphoreType.DMA,
            pltpu.SemaphoreType.DMA,
            pltpu.SemaphoreType.DMA,
            pltpu.SemaphoreType.DMA,
            pltpu.SemaphoreType.DMA,
            pltpu.SemaphoreType.DMA,
            pltpu.SemaphoreType.DMA,
            pltpu.SemaphoreType.DMA,
            pltpu.SemaphoreType.DMA,
            pltpu.SemaphoreType.DMA,
            pltpu.VMEM_SHARED((rows_sh, dh), jnp.float32),
            pltpu.VMEM_SHARED((n_nodes, dh), jnp.float32),
            pltpu.VMEM_SHARED((n_nodes, dh), jnp.float32),
        ],
        compiler_params=pltpu.CompilerParams(use_tc_tiling_on_sc=False),
    )


def _dis_col(dp_ref, n):
    deg = 1.0 + dp_ref[0][0:n, 0:1] + dp_ref[1][0:n, 0:1]
    return lax.rsqrt(deg)


def _tc1_body(x_ref, w1_ref, dp_ref, h1_ref):
    n, d = h1_ref.shape[0], w1_ref.shape[1]
    dis = _dis_col(dp_ref, n)
    h = jnp.dot(x_ref[...], w1_ref[...], preferred_element_type=jnp.float32)
    h1_ref[:, 0:d] = h * dis
    h1_ref[:, d:] = jnp.zeros((n, h1_ref.shape[1] - d), jnp.float32)


def _agg_total(lo_ref, hi_ref, h_ref, n, d):
    dh = d // 2
    return (
        jnp.concatenate(
            [lo_ref[0][0:n] + lo_ref[1][0:n], hi_ref[0][0:n] + hi_ref[1][0:n]],
            axis=1,
        )
        + h_ref[0:n, 0:d]
    )


def _tc2_body(lo_ref, hi_ref, h1_ref, dp_ref, w2_ref, b1_ref, h2_ref):
    n, d = h2_ref.shape[0], w2_ref.shape[1]
    dis = _dis_col(dp_ref, n)
    out1 = _agg_total(lo_ref, hi_ref, h1_ref, n, d) * dis + b1_ref[...]
    h2_ref[:, 0:d] = jnp.dot(out1, w2_ref[...], preferred_element_type=jnp.float32) * dis
    h2_ref[:, d:] = jnp.zeros((n, h2_ref.shape[1] - d), jnp.float32)


def _tc3_body(lo_ref, hi_ref, h2_ref, dp_ref, b2_ref, out_ref):
    n, d = out_ref.shape
    dis = _dis_col(dp_ref, n)
    out_ref[...] = _agg_total(lo_ref, hi_ref, h2_ref, n, d) * dis + b2_ref[...]


def kernel(x, edge_index, W1, b1, W2, b2):
    n, d_in = x.shape
    d_out = W1.shape[1]
    e = edge_index.shape[1]

    n_batches = 8 * (-(-e // (_NW * _LB * 8)))
    e_pad = _NW * _LB * n_batches
    rows_sh = _NS * _LB * (-(-(n + 1) // (_NS * _LB)))
    assert d_out % 16 == 0

    src = jnp.concatenate([edge_index[0], jnp.zeros((e_pad - e,), jnp.int32)]).reshape(-1, _LB)
    dst = jnp.concatenate([edge_index[1], jnp.full((e_pad - e,), n, jnp.int32)]).reshape(-1, _LB)

    dp = _deg_kernel_fn(n, n_batches, rows_sh)(dst).reshape(_NC, rows_sh, 16)

    dh = d_out // 2
    agg_raw = _agg_kernel_fn(n, d_out, n_batches, rows_sh)

    def agg(h128):
        lo, hi = agg_raw(h128, src, dst)
        return lo.reshape(_NC, rows_sh, dh), hi.reshape(_NC, rows_sh, dh)

    b1r = b1.reshape(1, d_out)
    b2r = b2.reshape(1, d_out)

    h1 = pl.pallas_call(
        _tc1_body,
        out_shape=jax.ShapeDtypeStruct((n, 128), jnp.float32),
    )(x, W1, dp)

    ap1_lo, ap1_hi = agg(h1)

    h2 = pl.pallas_call(
        _tc2_body,
        out_shape=jax.ShapeDtypeStruct((n, 128), jnp.float32),
    )(ap1_lo, ap1_hi, h1, dp, W2, b1r)

    ap2_lo, ap2_hi = agg(h2)

    out = pl.pallas_call(
        _tc3_body,
        out_shape=jax.ShapeDtypeStruct((n, d_out), jnp.float32),
    )(ap2_lo, ap2_hi, h2, dp, b2r)

    return out


# speedup vs baseline: 1.0510x; 1.0510x over previous
import functools

import jax
import jax.numpy as jnp
from jax import lax
from jax.experimental import pallas as pl
from jax.experimental.pallas import tpu as pltpu
from jax.experimental.pallas import tpu_sc as plsc

_NC = 2
_NS = 16
_NW = _NC * _NS
_LB = 128


def _deg_kernel_fn(n_nodes, n_batches, rows_sh):
    rows_per_sub = rows_sh // _NS
    zcopies = rows_per_sub // _LB

    def body(dst_flat, part_flat, dst_v, ones_v, zbuf, deg_sh):
        cid = lax.axis_index("c")
        sid = lax.axis_index("s")
        wid = sid * _NC + cid
        dst3 = dst_flat
        part = part_flat

        zero16 = jnp.zeros((16,), jnp.float32)
        one16 = jnp.ones((16,), jnp.float32)

        def fill(i, _):
            zbuf[i, 0:16] = zero16
            ones_v[i, 0:16] = one16
            return 0

        lax.fori_loop(0, _LB, fill, 0)

        for k in range(zcopies):
            pltpu.sync_copy(zbuf, deg_sh.at[pl.ds(sid * rows_per_sub + k * _LB, _LB)])
        plsc.subcore_barrier()

        pltpu.sync_copy(dst3.at[pl.ds(wid * n_batches, n_batches)], dst_v)

        def step(j, _):
            pltpu.sync_copy(ones_v, deg_sh.at[dst_v.at[j]], add=True)
            return 0

        lax.fori_loop(0, n_batches, step, 0)
        plsc.subcore_barrier()

        pltpu.sync_copy(
            deg_sh.at[pl.ds(sid * rows_per_sub, rows_per_sub)],
            part.at[pl.ds(cid * rows_sh + sid * rows_per_sub, rows_per_sub)],
        )

    return pl.kernel(
        body,
        out_type=jax.ShapeDtypeStruct((_NC * rows_sh, 16), jnp.float32),
        mesh=plsc.VectorSubcoreMesh(core_axis_name="c", subcore_axis_name="s"),
        scratch_types=[
            pltpu.VMEM((n_batches, _LB), jnp.int32),
            pltpu.VMEM((_LB, 16), jnp.float32),
            pltpu.VMEM((_LB, 16), jnp.float32),
            pltpu.VMEM_SHARED((rows_sh, 16), jnp.float32),
        ],
        compiler_params=pltpu.CompilerParams(use_tc_tiling_on_sc=False),
    )


def _agg_kernel_fn(n_nodes, d, n_batches, rows_sh):
    rows_per_sub = rows_sh // _NS
    zcopies = rows_per_sub // _LB

    nbuf = 8
    assert n_batches % nbuf == 0 and d % 32 == 0
    dh = d // 2
    n_per_sub = n_nodes // _NS

    def body(h128, src3, dst3, part_lo, part_hi, src_v, dst_v, rows_v,
             zbuf, sg0, sg1, sg2, sg3, sg4, sg5, sg6, sg7,
             agg_sh, hlo_sh, hhi_sh):
        cid = lax.axis_index("c")
        sid = lax.axis_index("s")
        wid = sid * _NC + cid
        sgs = (sg0, sg1, sg2, sg3, sg4, sg5, sg6, sg7)

        zero16 = jnp.zeros((16,), jnp.float32)

        def fill(i, _):
            for j in range(dh // 16):
                zbuf[i, pl.ds(j * 16, 16)] = zero16
            return 0

        lax.fori_loop(0, _LB, fill, 0)

        pltpu.sync_copy(
            h128.at[pl.ds(sid * n_per_sub, n_per_sub), pl.ds(0, dh)],
            hlo_sh.at[pl.ds(sid * n_per_sub, n_per_sub)],
        )
        pltpu.sync_copy(
            h128.at[pl.ds(sid * n_per_sub, n_per_sub), pl.ds(dh, dh)],
            hhi_sh.at[pl.ds(sid * n_per_sub, n_per_sub)],
        )

        pltpu.sync_copy(src3.at[pl.ds(wid * n_batches, n_batches)], src_v)
        pltpu.sync_copy(dst3.at[pl.ds(wid * n_batches, n_batches)], dst_v)

        def phase(h_sh, part):
            for k in range(zcopies):
                pltpu.sync_copy(
                    zbuf, agg_sh.at[pl.ds(sid * rows_per_sub + k * _LB, _LB)]
                )
            plsc.subcore_barrier()

            for b in range(nbuf):
                pltpu.async_copy(h_sh.at[src_v.at[b]], rows_v.at[b], sgs[b])

            def group(jj, _):
                for b in range(nbuf):
                    j = jj * nbuf + b
                    pltpu.make_async_copy(
                        h_sh.at[src_v.at[j]], rows_v.at[b], sgs[b]
                    ).wait()
                    pltpu.sync_copy(rows_v.at[b], agg_sh.at[dst_v.at[j]], add=True)

                    @pl.when(j + nbuf < n_batches)
                    def _():
                        pltpu.async_copy(
                            h_sh.at[src_v.at[j + nbuf]], rows_v.at[b], sgs[b]
                        )

                return 0

            lax.fori_loop(0, n_batches // nbuf, group, 0)
            plsc.subcore_barrier()

            pltpu.sync_copy(
                agg_sh.at[pl.ds(sid * rows_per_sub, rows_per_sub)],
                part.at[pl.ds(cid * rows_sh + sid * rows_per_sub, rows_per_sub)],
            )

        phase(hlo_sh, part_lo)
        phase(hhi_sh, part_hi)

    return pl.kernel(
        body,
        out_type=(
            jax.ShapeDtypeStruct((_NC * rows_sh, dh), jnp.float32),
            jax.ShapeDtypeStruct((_NC * rows_sh, dh), jnp.float32),
        ),
        mesh=plsc.VectorSubcoreMesh(core_axis_name="c", subcore_axis_name="s"),
        scratch_types=[
            pltpu.VMEM((n_batches, _LB), jnp.int32),
            pltpu.VMEM((n_batches, _LB), jnp.int32),
            pltpu.VMEM((nbuf, _LB, dh), jnp.float32),
            pltpu.VMEM((_LB, dh), jnp.float32),
            pltpu.SemaphoreType.DMA,
            pltpu.SemaphoreType.DMA,
            pltpu.SemaphoreType.DMA,
            pltpu.SemaphoreType.DMA,
            pltpu.SemaphoreType.DMA,
            pltpu.SemaphoreType.DMA,
            pltpu.SemaphoreType.DMA,
            pltpu.SemaphoreType.DMA,
            pltpu.VMEM_SHARED((rows_sh, dh), jnp.float32),
            pltpu.VMEM_SHARED((n_nodes, dh), jnp.float32),
            pltpu.VMEM_SHARED((n_nodes, dh), jnp.float32),
        ],
        compiler_params=pltpu.CompilerParams(use_tc_tiling_on_sc=False),
    )


def _dis_col(dp_ref, n):
    deg = 1.0 + dp_ref[0][0:n, 0:1] + dp_ref[1][0:n, 0:1]
    return lax.rsqrt(deg)


def _tc1_body(x_ref, w1_ref, dp_ref, h1_ref):
    n, d = h1_ref.shape[0], w1_ref.shape[1]
    dis = _dis_col(dp_ref, n)
    h = jnp.dot(x_ref[...], w1_ref[...], preferred_element_type=jnp.float32)
    h1_ref[:, 0:d] = h * dis
    h1_ref[:, d:] = jnp.zeros((n, h1_ref.shape[1] - d), jnp.float32)


def _agg_total(lo_ref, hi_ref, h_ref, n, d):
    dh = d // 2
    return (
        jnp.concatenate(
            [lo_ref[0][0:n] + lo_ref[1][0:n], hi_ref[0][0:n] + hi_ref[1][0:n]],
            axis=1,
        )
        + h_ref[0:n, 0:d]
    )


def _tc2_body(lo_ref, hi_ref, h1_ref, dp_ref, w2_ref, b1_ref, h2_ref):
    n, d = h2_ref.shape[0], w2_ref.shape[1]
    dis = _dis_col(dp_ref, n)
    out1 = _agg_total(lo_ref, hi_ref, h1_ref, n, d) * dis + b1_ref[...]
    h2_ref[:, 0:d] = jnp.dot(out1, w2_ref[...], preferred_element_type=jnp.float32) * dis
    h2_ref[:, d:] = jnp.zeros((n, h2_ref.shape[1] - d), jnp.float32)


def _tc3_body(lo_ref, hi_ref, h2_ref, dp_ref, b2_ref, out_ref):
    n, d = out_ref.shape
    dis = _dis_col(dp_ref, n)
    out_ref[...] = _agg_total(lo_ref, hi_ref, h2_ref, n, d) * dis + b2_ref[...]


def kernel(x, edge_index, W1, b1, W2, b2):
    n, d_in = x.shape
    d_out = W1.shape[1]
    e = edge_index.shape[1]

    n_batches = 8 * (-(-e // (_NW * _LB * 8)))
    e_pad = _NW * _LB * n_batches
    rows_sh = _NS * _LB * (-(-(n + 1) // (_NS * _LB)))
    assert d_out % 16 == 0

    src = jnp.concatenate([edge_index[0], jnp.zeros((e_pad - e,), jnp.int32)]).reshape(-1, _LB)
    dst = jnp.concatenate([edge_index[1], jnp.full((e_pad - e,), n, jnp.int32)]).reshape(-1, _LB)

    dp = _deg_kernel_fn(n, n_batches, rows_sh)(dst).reshape(_NC, rows_sh, 16)

    dh = d_out // 2
    agg_raw = _agg_kernel_fn(n, d_out, n_batches, rows_sh)

    def agg(h128):
        lo, hi = agg_raw(h128, src, dst)
        return lo.reshape(_NC, rows_sh, dh), hi.reshape(_NC, rows_sh, dh)

    b1r = b1.reshape(1, d_out)
    b2r = b2.reshape(1, d_out)

    h1 = pl.pallas_call(
        _tc1_body,
        out_shape=jax.ShapeDtypeStruct((n, 128), jnp.float32),
    )(x, W1, dp)

    ap1_lo, ap1_hi = agg(h1)

    h2 = pl.pallas_call(
        _tc2_body,
        out_shape=jax.ShapeDtypeStruct((n, 128), jnp.float32),
    )(ap1_lo, ap1_hi, h1, dp, W2, b1r)

    ap2_lo, ap2_hi = agg(h2)

    out = pl.pallas_call(
        _tc3_body,
        out_shape=jax.ShapeDtypeStruct((n, d_out), jnp.float32),
    )(ap2_lo, ap2_hi, h2, dp, b2r)

    return out
